# baseline (speedup 1.0000x reference)
import functools

import jax
import jax.numpy as jnp
from jax import lax
from jax.experimental import pallas as pl
from jax.experimental.pallas import tpu as pltpu
from jax.experimental.pallas import tpu_sc as plsc

_NC, _NS = 2, 16
_NW = _NC * _NS
_CHUNK = 128
_DD = 16


_NBUF = 6


def _make_sc_gather(rows_pad: int, d: int):
    chunks_per_w = rows_pad // (_NW * _CHUNK)
    nbuf = min(_NBUF, chunks_per_w)
    mesh = plsc.VectorSubcoreMesh(core_axis_name="c", subcore_axis_name="s")

    @functools.partial(
        pl.kernel,
        mesh=mesh,
        out_type=jax.ShapeDtypeStruct((rows_pad, d), jnp.float32),
        scratch_types=[
            pltpu.VMEM((chunks_per_w, _CHUNK), jnp.int32),
            pltpu.VMEM((nbuf, _CHUNK, d), jnp.float32),
            pltpu.SemaphoreType.DMA((nbuf,)),
            pltpu.SemaphoreType.DMA((nbuf,)),
        ],
    )
    def gather_kernel(table_hbm, idx_hbm, out_hbm, idx_v, rows_v, sem_g, sem_s):
        w = lax.axis_index("c") * _NS + lax.axis_index("s")
        base = w * chunks_per_w
        pltpu.sync_copy(idx_hbm.at[w], idx_v)

        def gather_copy(k, b):
            return pltpu.make_async_copy(
                table_hbm.at[idx_v.at[k]], rows_v.at[b], sem_g.at[b]
            )

        def store_copy(k, b):
            return pltpu.make_async_copy(
                rows_v.at[b],
                out_hbm.at[pl.ds((base + k) * _CHUNK, _CHUNK)],
                sem_s.at[b],
            )

        for k in range(nbuf):
            gather_copy(k, k).start()
        store_waited = set()
        for k in range(chunks_per_w):
            gather_copy(k, k % nbuf).wait()
            store_copy(k, k % nbuf).start()
            pk = k - 1
            if pk >= 0 and pk + nbuf < chunks_per_w:
                store_copy(pk, pk % nbuf).wait()
                store_waited.add(pk)
                gather_copy(pk + nbuf, pk % nbuf).start()
        for k in range(chunks_per_w):
            if k not in store_waited:
                store_copy(k, k % nbuf).wait()

    return gather_kernel


def _tc_body(z_ref, x_ref, a2_ref, er_ref, c_ref, acc_ref, o_ref):
    del acc_ref
    b, m, d = z_ref.shape
    z3 = z_ref[...]
    zf = z3.reshape(b * m, d)
    ecol = jnp.dot(zf, a2_ref[...], preferred_element_type=jnp.float32)
    e3 = ecol.reshape(b, m, 1)
    e3 = e3 - jnp.max(e3, axis=1, keepdims=True)
    ex = jnp.exp(e3)
    attw3 = ex / jnp.sum(ex, axis=1, keepdims=True)
    srep = jnp.dot(zf, er_ref[...], preferred_element_type=jnp.float32)
    y3 = (zf * srep).reshape(b, m, d) * attw3
    u = jnp.sum(y3, axis=1) + x_ref[...]
    nrep = jnp.dot(u * u, er_ref[...], preferred_element_type=jnp.float32)
    inv = 1.0 / jnp.maximum(jnp.sqrt(nrep), 1e-12)
    o_ref[...] = jnp.where(c_ref[...] > 0.0, u * inv, u)


def kernel(x, neighbors, att, max_iter):
    n, d = x.shape
    m = neighbors.shape[0] // n
    gran = _NW * _CHUNK

    table = jnp.concatenate([x, jnp.zeros((1, d), x.dtype)], axis=0)
    a2 = att[d:, :]
    ii = lax.broadcasted_iota(jnp.int32, (d, d), 0) // _DD
    jj = lax.broadcasted_iota(jnp.int32, (d, d), 1) // _DD
    erep = (ii == jj).astype(jnp.float32)
    cond = (jnp.asarray(max_iter) > 3).astype(jnp.float32).reshape(1, 1)

    sizes = [1600] * 5 + [2000]
    bsz = 400
    gather_fns = {}

    def tc_call(z3_p, node0, nblocks, acc):
        b0 = node0 // bsz
        return pl.pallas_call(
            _tc_body,
            grid=(nblocks,),
            in_specs=[
                pl.BlockSpec((bsz, m, d), lambda i: (i, 0, 0)),
                pl.BlockSpec((bsz, d), lambda i, b0=b0: (b0 + i, 0)),
                pl.BlockSpec((d, 1), lambda i: (0, 0)),
                pl.BlockSpec((d, d), lambda i: (0, 0)),
                pl.BlockSpec((1, 1), lambda i: (0, 0)),
                pl.BlockSpec(memory_space=pl.ANY),
            ],
            out_specs=pl.BlockSpec(
                (bsz, d), lambda i, b0=b0: (b0 + i, 0)
            ),
            out_shape=jax.ShapeDtypeStruct((n, d), jnp.float32),
            input_output_aliases={5: 0},
        )(z3_p, x, a2, erep, cond, acc)

    acc = jnp.zeros((n, d), jnp.float32)
    node0 = 0
    for nodes_p in sizes:
        rows_p = nodes_p * m
        rows_pad_p = ((rows_p + gran - 1) // gran) * gran
        if rows_pad_p not in gather_fns:
            gather_fns[rows_pad_p] = _make_sc_gather(rows_pad_p, d)
        nb_p = lax.dynamic_slice_in_dim(neighbors, node0 * m, rows_p)
        pad_idx = jnp.arange(rows_pad_p - rows_p, dtype=neighbors.dtype)
        idx2 = jnp.concatenate([nb_p, pad_idx]).reshape(
            _NW, rows_pad_p // (_NW * _CHUNK), _CHUNK
        )
        zg = gather_fns[rows_pad_p](table, idx2)
        z3_p = zg.reshape(rows_pad_p // m, m, d)
        acc = tc_call(z3_p, node0, nodes_p // bsz, acc)
        node0 += nodes_p
    return acc


# speedup vs baseline: 1.0276x; 1.0276x over previous
import functools

import jax
import jax.numpy as jnp
from jax import lax
from jax.experimental import pallas as pl
from jax.experimental.pallas import tpu as pltpu
from jax.experimental.pallas import tpu_sc as plsc

_NC, _NS = 2, 16
_NW = _NC * _NS
_CHUNK = 128
_DD = 16


_NBUF = 7


def _make_sc_gather(rows_pad: int, d: int):
    chunks_per_w = rows_pad // (_NW * _CHUNK)
    nbuf = min(_NBUF, chunks_per_w)
    mesh = plsc.VectorSubcoreMesh(core_axis_name="c", subcore_axis_name="s")

    @functools.partial(
        pl.kernel,
        mesh=mesh,
        out_type=jax.ShapeDtypeStruct((rows_pad, d), jnp.float32),
        scratch_types=[
            pltpu.VMEM((chunks_per_w, _CHUNK), jnp.int32),
            pltpu.VMEM((nbuf, _CHUNK, d), jnp.float32),
            pltpu.SemaphoreType.DMA((nbuf,)),
            pltpu.SemaphoreType.DMA((nbuf,)),
        ],
    )
    def gather_kernel(table_hbm, idx_hbm, out_hbm, idx_v, rows_v, sem_g, sem_s):
        w = lax.axis_index("c") * _NS + lax.axis_index("s")
        base = w * chunks_per_w
        pltpu.sync_copy(idx_hbm.at[w], idx_v)

        def gather_copy(k, b):
            return pltpu.make_async_copy(
                table_hbm.at[idx_v.at[k]], rows_v.at[b], sem_g.at[b]
            )

        def store_copy(k, b):
            return pltpu.make_async_copy(
                rows_v.at[b],
                out_hbm.at[pl.ds((base + k) * _CHUNK, _CHUNK)],
                sem_s.at[b],
            )

        for k in range(nbuf):
            gather_copy(k, k).start()
        store_waited = set()
        for k in range(chunks_per_w):
            gather_copy(k, k % nbuf).wait()
            store_copy(k, k % nbuf).start()
            pk = k - 1
            if pk >= 0 and pk + nbuf < chunks_per_w:
                store_copy(pk, pk % nbuf).wait()
                store_waited.add(pk)
                gather_copy(pk + nbuf, pk % nbuf).start()
        for k in range(chunks_per_w):
            if k not in store_waited:
                store_copy(k, k % nbuf).wait()

    return gather_kernel


def _tc_body(z_ref, x_ref, a2_ref, er_ref, c_ref, acc_ref, o_ref):
    del acc_ref
    b, m, d = z_ref.shape
    z3 = z_ref[...]
    zf = z3.reshape(b * m, d)
    ecol = jnp.dot(zf, a2_ref[...], preferred_element_type=jnp.float32)
    e3 = ecol.reshape(b, m, 1)
    e3 = e3 - jnp.max(e3, axis=1, keepdims=True)
    ex = jnp.exp(e3)
    attw3 = ex / jnp.sum(ex, axis=1, keepdims=True)
    srep = jnp.dot(zf, er_ref[...], preferred_element_type=jnp.float32)
    y3 = (zf * srep).reshape(b, m, d) * attw3
    u = jnp.sum(y3, axis=1) + x_ref[...]
    nrep = jnp.dot(u * u, er_ref[...], preferred_element_type=jnp.float32)
    inv = 1.0 / jnp.maximum(jnp.sqrt(nrep), 1e-12)
    o_ref[...] = jnp.where(c_ref[...] > 0.0, u * inv, u)


def kernel(x, neighbors, att, max_iter):
    n, d = x.shape
    m = neighbors.shape[0] // n
    gran = _NW * _CHUNK

    table = jnp.concatenate([x, jnp.zeros((1, d), x.dtype)], axis=0)
    a2 = att[d:, :]
    ii = lax.broadcasted_iota(jnp.int32, (d, d), 0) // _DD
    jj = lax.broadcasted_iota(jnp.int32, (d, d), 1) // _DD
    erep = (ii == jj).astype(jnp.float32)
    cond = (jnp.asarray(max_iter) > 3).astype(jnp.float32).reshape(1, 1)

    sizes = [2000] * 5
    bsz = 400
    gather_fns = {}

    def tc_call(z3_p, node0, nblocks, acc):
        b0 = node0 // bsz
        return pl.pallas_call(
            _tc_body,
            grid=(nblocks,),
            in_specs=[
                pl.BlockSpec((bsz, m, d), lambda i: (i, 0, 0)),
                pl.BlockSpec((bsz, d), lambda i, b0=b0: (b0 + i, 0)),
                pl.BlockSpec((d, 1), lambda i: (0, 0)),
                pl.BlockSpec((d, d), lambda i: (0, 0)),
                pl.BlockSpec((1, 1), lambda i: (0, 0)),
                pl.BlockSpec(memory_space=pl.ANY),
            ],
            out_specs=pl.BlockSpec(
                (bsz, d), lambda i, b0=b0: (b0 + i, 0)
            ),
            out_shape=jax.ShapeDtypeStruct((n, d), jnp.float32),
            input_output_aliases={5: 0},
        )(z3_p, x, a2, erep, cond, acc)

    acc = jnp.zeros((n, d), jnp.float32)
    node0 = 0
    for nodes_p in sizes:
        rows_p = nodes_p * m
        rows_pad_p = ((rows_p + gran - 1) // gran) * gran
        if rows_pad_p not in gather_fns:
            gather_fns[rows_pad_p] = _make_sc_gather(rows_pad_p, d)
        nb_p = lax.dynamic_slice_in_dim(neighbors, node0 * m, rows_p)
        pad_idx = jnp.arange(rows_pad_p - rows_p, dtype=neighbors.dtype)
        idx2 = jnp.concatenate([nb_p, pad_idx]).reshape(
            _NW, rows_pad_p // (_NW * _CHUNK), _CHUNK
        )
        zg = gather_fns[rows_pad_p](table, idx2)
        z3_p = zg.reshape(rows_pad_p // m, m, d)
        acc = tc_call(z3_p, node0, nodes_p // bsz, acc)
        node0 += nodes_p
    return acc
